# Initial kernel scaffold; baseline (speedup 1.0000x reference)
import jax
import jax.numpy as jnp
from jax.experimental import pallas as pl


def kernel(h, adj, W, a):
    raise NotImplementedError("write your pallas kernel here")


# speedup vs baseline: 1962.1659x; 1962.1659x over previous
import jax
import jax.numpy as jnp
from jax.experimental import pallas as pl

ALPHA = 0.2
EPS = 1e-16


def _gat_block_kernel(h_ref, hblk_ref, adjt_ref, w_ref, a_ref, out_ref):
    dout = w_ref.shape[1]
    wh = jnp.dot(h_ref[...], w_ref[...], preferred_element_type=jnp.float32)
    wh_blk = jnp.dot(hblk_ref[...], w_ref[...], preferred_element_type=jnp.float32)
    s1 = jnp.dot(wh, a_ref[:dout, :], preferred_element_type=jnp.float32)
    s2_blk = jnp.dot(wh_blk, a_ref[dout:, :], preferred_element_type=jnp.float32)
    b = s2_blk + s1.T
    b = jnp.where(b >= 0.0, b, ALPHA * b)
    mask = adjt_ref[...] > 0.0
    bm = jnp.where(mask, b, -jnp.inf)
    m = jnp.max(bm, axis=1, keepdims=True)
    m = jnp.where(jnp.isfinite(m), m, 0.0)
    p = jnp.where(mask, jnp.exp(b - m), 0.0)
    denom = jnp.sum(p, axis=1, keepdims=True) + EPS
    att = p / denom
    hp = jnp.dot(att, wh, preferred_element_type=jnp.float32)
    out_ref[...] = jnp.where(hp > 0.0, hp, jnp.exp(hp) - 1.0)


def kernel(h, adj, W, a):
    N, din = h.shape
    dout = W.shape[1]
    blk = 128
    grid = N // blk
    adjt = adj.T
    return pl.pallas_call(
        _gat_block_kernel,
        grid=(grid,),
        in_specs=[
            pl.BlockSpec((N, din), lambda i: (0, 0)),
            pl.BlockSpec((blk, din), lambda i: (i, 0)),
            pl.BlockSpec((blk, N), lambda i: (i, 0)),
            pl.BlockSpec((din, dout), lambda i: (0, 0)),
            pl.BlockSpec((2 * dout, 1), lambda i: (0, 0)),
        ],
        out_specs=pl.BlockSpec((blk, dout), lambda i: (i, 0)),
        out_shape=jax.ShapeDtypeStruct((N, dout), jnp.float32),
    )(h, h, adjt, W, a)


# speedup vs baseline: 2792.1588x; 1.4230x over previous
import jax
import jax.numpy as jnp
from jax.experimental import pallas as pl

ALPHA = 0.2
EPS = 1e-16


def _gat_block_kernel(h_ref, hblk_ref, adj_ref, w_ref, a_ref, out_ref):
    dout = w_ref.shape[1]
    wh = jnp.dot(h_ref[...], w_ref[...], preferred_element_type=jnp.float32)
    wh_blk = jnp.dot(hblk_ref[...], w_ref[...], preferred_element_type=jnp.float32)
    s1 = jnp.dot(wh, a_ref[:dout, :], preferred_element_type=jnp.float32)
    s2_blk = jnp.dot(wh_blk, a_ref[dout:, :], preferred_element_type=jnp.float32)
    b = s1 + s2_blk.T
    b = jnp.where(b >= 0.0, b, ALPHA * b)
    mask = adj_ref[...] > 0.0
    bm = jnp.where(mask, b, -jnp.inf)
    m = jnp.max(bm, axis=0, keepdims=True)
    m = jnp.where(jnp.isfinite(m), m, 0.0)
    p = jnp.where(mask, jnp.exp(b - m), 0.0)
    denom = jnp.sum(p, axis=0, keepdims=True) + EPS
    att = p / denom
    hp = jax.lax.dot_general(
        att, wh, (((0,), (0,)), ((), ())),
        preferred_element_type=jnp.float32,
    )
    out_ref[...] = jnp.where(hp > 0.0, hp, jnp.exp(hp) - 1.0)


def kernel(h, adj, W, a):
    N, din = h.shape
    dout = W.shape[1]
    blk = 128
    grid = N // blk
    return pl.pallas_call(
        _gat_block_kernel,
        grid=(grid,),
        in_specs=[
            pl.BlockSpec((N, din), lambda i: (0, 0)),
            pl.BlockSpec((blk, din), lambda i: (i, 0)),
            pl.BlockSpec((N, blk), lambda i: (0, i)),
            pl.BlockSpec((din, dout), lambda i: (0, 0)),
            pl.BlockSpec((2 * dout, 1), lambda i: (0, 0)),
        ],
        out_specs=pl.BlockSpec((blk, dout), lambda i: (i, 0)),
        out_shape=jax.ShapeDtypeStruct((N, dout), jnp.float32),
    )(h, h, adj, W, a)


# speedup vs baseline: 3153.1928x; 1.1293x over previous
import jax
import jax.numpy as jnp
from jax.experimental import pallas as pl
from jax.experimental.pallas import tpu as pltpu

ALPHA = 0.2
EPS = 1e-16


def _gat_block_kernel(h_ref, adj_ref, w_ref, a_ref, out_ref,
                      wh_ref, s1_ref, s2t_ref):
    dout = w_ref.shape[1]
    blk = adj_ref.shape[1]

    @pl.when(pl.program_id(0) == 0)
    def _():
        wh0 = jnp.dot(h_ref[...], w_ref[...], preferred_element_type=jnp.float32)
        wh_ref[...] = wh0
        s1_ref[...] = jnp.dot(wh0, a_ref[:dout, :],
                              preferred_element_type=jnp.float32)
        s2t_ref[...] = jnp.dot(wh0, a_ref[dout:, :],
                               preferred_element_type=jnp.float32).T

    j0 = pl.program_id(0) * blk
    wh = wh_ref[...]
    s1 = s1_ref[...]
    s2_blk = s2t_ref[:, pl.ds(j0, blk)]
    b = s1 + s2_blk
    b = jnp.where(b >= 0.0, b, ALPHA * b)
    mask = adj_ref[...] > 0.0
    bm = jnp.where(mask, b, -jnp.inf)
    m = jnp.max(bm, axis=0, keepdims=True)
    m = jnp.where(jnp.isfinite(m), m, 0.0)
    p = jnp.where(mask, jnp.exp(b - m), 0.0)
    denom = jnp.sum(p, axis=0, keepdims=True) + EPS
    att = p / denom
    hp = jax.lax.dot_general(
        att, wh, (((0,), (0,)), ((), ())),
        preferred_element_type=jnp.float32,
    )
    out_ref[...] = jnp.where(hp > 0.0, hp, jnp.exp(hp) - 1.0)


def kernel(h, adj, W, a):
    N, din = h.shape
    dout = W.shape[1]
    blk = 128
    grid = N // blk
    return pl.pallas_call(
        _gat_block_kernel,
        grid=(grid,),
        in_specs=[
            pl.BlockSpec((N, din), lambda i: (0, 0)),
            pl.BlockSpec((N, blk), lambda i: (0, i)),
            pl.BlockSpec((din, dout), lambda i: (0, 0)),
            pl.BlockSpec((2 * dout, 1), lambda i: (0, 0)),
        ],
        out_specs=pl.BlockSpec((blk, dout), lambda i: (i, 0)),
        out_shape=jax.ShapeDtypeStruct((N, dout), jnp.float32),
        scratch_shapes=[
            pltpu.VMEM((N, dout), jnp.float32),
            pltpu.VMEM((N, 1), jnp.float32),
            pltpu.VMEM((1, N), jnp.float32),
        ],
    )(h, adj, W, a)


# speedup vs baseline: 3310.8547x; 1.0500x over previous
import jax
import jax.numpy as jnp
from jax.experimental import pallas as pl
from jax.experimental.pallas import tpu as pltpu

ALPHA = 0.2
EPS = 1e-16


def _gat_block_kernel(h_ref, adj_ref, w_ref, a_ref, out_ref,
                      wh_ref, s1_ref, s2t_ref):
    dout = w_ref.shape[1]
    blk = adj_ref.shape[1]

    @pl.when(pl.program_id(0) == 0)
    def _():
        wh0 = jnp.dot(h_ref[...], w_ref[...], preferred_element_type=jnp.float32)
        wh_ref[...] = wh0
        s1_ref[...] = jnp.dot(wh0, a_ref[:dout, :],
                              preferred_element_type=jnp.float32)
        s2t_ref[...] = jnp.dot(wh0, a_ref[dout:, :],
                               preferred_element_type=jnp.float32).T

    j0 = pl.program_id(0) * blk
    wh = wh_ref[...]
    s1 = s1_ref[...]
    s2_blk = s2t_ref[:, pl.ds(j0, blk)]
    b = s1 + s2_blk
    b = jnp.where(b >= 0.0, b, ALPHA * b)
    bm = jnp.where(adj_ref[...] > 0.0, b, -jnp.inf)
    m = jnp.max(bm, axis=0, keepdims=True)
    m = jnp.where(jnp.isfinite(m), m, 0.0)
    p = jnp.exp(bm - m)
    denom = jnp.sum(p, axis=0, keepdims=True) + EPS
    hp = jax.lax.dot_general(
        p, wh, (((0,), (0,)), ((), ())),
        preferred_element_type=jnp.float32,
    ) * (1.0 / denom).T
    out_ref[...] = jnp.where(hp > 0.0, hp, jnp.exp(hp) - 1.0)


def kernel(h, adj, W, a):
    N, din = h.shape
    dout = W.shape[1]
    blk = 128
    grid = N // blk
    return pl.pallas_call(
        _gat_block_kernel,
        grid=(grid,),
        in_specs=[
            pl.BlockSpec((N, din), lambda i: (0, 0)),
            pl.BlockSpec((N, blk), lambda i: (0, i)),
            pl.BlockSpec((din, dout), lambda i: (0, 0)),
            pl.BlockSpec((2 * dout, 1), lambda i: (0, 0)),
        ],
        out_specs=pl.BlockSpec((blk, dout), lambda i: (i, 0)),
        out_shape=jax.ShapeDtypeStruct((N, dout), jnp.float32),
        scratch_shapes=[
            pltpu.VMEM((N, dout), jnp.float32),
            pltpu.VMEM((N, 1), jnp.float32),
            pltpu.VMEM((1, N), jnp.float32),
        ],
    )(h, adj, W, a)


# speedup vs baseline: 4258.5872x; 1.2863x over previous
import jax
import jax.numpy as jnp
from jax.experimental import pallas as pl
from jax.experimental.pallas import tpu as pltpu

ALPHA = 0.2
EPS = 1e-16


def _gat_block_kernel(h_ref, adj_ref, w_ref, a_ref, out_ref,
                      wh_ref, s1_ref, s2t_ref):
    dout = w_ref.shape[1]
    blk = adj_ref.shape[1]

    @pl.when(pl.program_id(0) == 0)
    def _():
        wh0 = jnp.dot(h_ref[...], w_ref[...], preferred_element_type=jnp.float32)
        wh_ref[...] = wh0
        s1_ref[...] = jnp.dot(wh0, a_ref[:dout, :],
                              preferred_element_type=jnp.float32)
        s2t_ref[...] = jnp.dot(wh0, a_ref[dout:, :],
                               preferred_element_type=jnp.float32).T

    j0 = pl.program_id(0) * blk
    wh = wh_ref[...]
    s1 = s1_ref[...]
    s2_blk = s2t_ref[:, pl.ds(j0, blk)]
    b = s1 + s2_blk
    b = jnp.where(b >= 0.0, b, ALPHA * b)
    bm = jnp.where(adj_ref[...] > 0.0, b, -jnp.inf)
    m = jnp.max(bm, axis=0, keepdims=True)
    m = jnp.where(jnp.isfinite(m), m, 0.0)
    p = jnp.exp(bm - m)
    denom = jnp.sum(p, axis=0, keepdims=True) + EPS
    hp = jax.lax.dot_general(
        p, wh, (((0,), (0,)), ((), ())),
        preferred_element_type=jnp.float32,
    ) * (1.0 / denom).T
    out_ref[...] = jnp.where(hp > 0.0, hp, jnp.exp(hp) - 1.0)


def kernel(h, adj, W, a):
    N, din = h.shape
    dout = W.shape[1]
    blk = 256
    grid = N // blk
    return pl.pallas_call(
        _gat_block_kernel,
        grid=(grid,),
        in_specs=[
            pl.BlockSpec((N, din), lambda i: (0, 0)),
            pl.BlockSpec((N, blk), lambda i: (0, i)),
            pl.BlockSpec((din, dout), lambda i: (0, 0)),
            pl.BlockSpec((2 * dout, 1), lambda i: (0, 0)),
        ],
        out_specs=pl.BlockSpec((blk, dout), lambda i: (i, 0)),
        out_shape=jax.ShapeDtypeStruct((N, dout), jnp.float32),
        scratch_shapes=[
            pltpu.VMEM((N, dout), jnp.float32),
            pltpu.VMEM((N, 1), jnp.float32),
            pltpu.VMEM((1, N), jnp.float32),
        ],
    )(h, adj, W, a)


# speedup vs baseline: 4557.1508x; 1.0701x over previous
import jax
import jax.numpy as jnp
from jax.experimental import pallas as pl
from jax.experimental.pallas import tpu as pltpu

ALPHA = 0.2
EPS = 1e-16


def _gat_block_kernel(h_ref, adj_ref, w_ref, a_ref, out_ref,
                      wh_ref, s1_ref, s2t_ref):
    dout = w_ref.shape[1]
    blk = adj_ref.shape[1]

    @pl.when(pl.program_id(0) == 0)
    def _():
        wh0 = jnp.dot(h_ref[...], w_ref[...], preferred_element_type=jnp.float32)
        wh_ref[...] = wh0
        s1_ref[...] = jnp.dot(wh0, a_ref[:dout, :],
                              preferred_element_type=jnp.float32)
        s2t_ref[...] = jnp.dot(wh0, a_ref[dout:, :],
                               preferred_element_type=jnp.float32).T

    j0 = pl.program_id(0) * blk
    wh = wh_ref[...]
    s1 = s1_ref[...]
    s2_blk = s2t_ref[:, pl.ds(j0, blk)]
    b = s1 + s2_blk
    b = jnp.where(b >= 0.0, b, ALPHA * b)
    bm = jnp.where(adj_ref[...] > 0.0, b, -jnp.inf)
    m = jnp.max(bm, axis=0, keepdims=True)
    m = jnp.where(jnp.isfinite(m), m, 0.0)
    p = jnp.exp(bm - m)
    denom = jnp.sum(p, axis=0, keepdims=True) + EPS
    hp = jax.lax.dot_general(
        p, wh, (((0,), (0,)), ((), ())),
        preferred_element_type=jnp.float32,
    ) * (1.0 / denom).T
    out_ref[...] = jnp.where(hp > 0.0, hp, jnp.exp(hp) - 1.0)


def kernel(h, adj, W, a):
    N, din = h.shape
    dout = W.shape[1]
    blk = 512
    grid = N // blk
    return pl.pallas_call(
        _gat_block_kernel,
        grid=(grid,),
        in_specs=[
            pl.BlockSpec((N, din), lambda i: (0, 0)),
            pl.BlockSpec((N, blk), lambda i: (0, i)),
            pl.BlockSpec((din, dout), lambda i: (0, 0)),
            pl.BlockSpec((2 * dout, 1), lambda i: (0, 0)),
        ],
        out_specs=pl.BlockSpec((blk, dout), lambda i: (i, 0)),
        out_shape=jax.ShapeDtypeStruct((N, dout), jnp.float32),
        scratch_shapes=[
            pltpu.VMEM((N, dout), jnp.float32),
            pltpu.VMEM((N, 1), jnp.float32),
            pltpu.VMEM((1, N), jnp.float32),
        ],
    )(h, adj, W, a)
